# baseline (speedup 1.0000x reference)
import functools

import jax
import jax.numpy as jnp
from jax import lax
from jax.experimental import pallas as pl
from jax.experimental.pallas import tpu as pltpu
from jax.experimental.pallas import tpu_sc as plsc

_B = 16384
_L = 16


@functools.lru_cache(maxsize=None)
def _build(table_len: int):
    info = plsc.get_sparse_core_info()
    nc, ns = 1, info.num_subcores
    nw = nc * ns
    b_per_w = _B // nw

    mesh = plsc.VectorSubcoreMesh(core_axis_name="c", subcore_axis_name="s",
                                  num_cores=nc)

    @functools.partial(
        pl.kernel,
        mesh=mesh,
        out_type=jax.ShapeDtypeStruct((_B,), jnp.float32),
        compiler_params=pltpu.CompilerParams(
            needs_layout_passes=False,
            skip_device_barrier=True,
            disable_bounds_checks=True,
            disable_semaphore_checks=True,
        ),
        scratch_types=[
            pltpu.VMEM((table_len,), jnp.float32),
            pltpu.VMEM((b_per_w,), jnp.int32),
            pltpu.VMEM((b_per_w,), jnp.float32),
            pltpu.SemaphoreType.DMA,
            pltpu.SemaphoreType.DMA,
        ],
    )
    def k(table_hbm, idx_hbm, out_hbm, table_v, idx_v, vals_v, sem_t, sem_i):
        wid = lax.axis_index("s") * nc + lax.axis_index("c")
        base = wid * b_per_w
        half = b_per_w // 2
        cp_t = pltpu.async_copy(table_hbm, table_v, sem_t)
        cp_i0 = pltpu.async_copy(idx_hbm.at[pl.ds(base, half)],
                                 idx_v.at[pl.ds(0, half)], sem_i)
        cp_i1 = pltpu.async_copy(idx_hbm.at[pl.ds(base + half, half)],
                                 idx_v.at[pl.ds(half, half)], sem_i)
        cp_i0.wait()
        cp_t.wait()

        @plsc.parallel_loop(0, half, _L, unroll=8)
        def _(off):
            idx16 = idx_v[pl.ds(off, _L)]
            vals_v[pl.ds(off, _L)] = plsc.load_gather(table_v, [idx16])
        cp_o0 = pltpu.async_copy(vals_v.at[pl.ds(0, half)],
                                 out_hbm.at[pl.ds(base, half)], sem_t)
        cp_i1.wait()

        @plsc.parallel_loop(half, b_per_w, _L, unroll=8)
        def _(off):
            idx16 = idx_v[pl.ds(off, _L)]
            vals_v[pl.ds(off, _L)] = plsc.load_gather(table_v, [idx16])
        cp_o1 = pltpu.async_copy(vals_v.at[pl.ds(half, half)],
                                 out_hbm.at[pl.ds(base + half, half)], sem_i)
        cp_o0.wait()
        cp_o1.wait()

    return k


def kernel(betas, t_int):
    return _build(betas.shape[0])(betas.astype(jnp.float32),
                                  t_int.astype(jnp.int32))


# speedup vs baseline: 1.0045x; 1.0045x over previous
import functools

import jax
import jax.numpy as jnp
from jax import lax
from jax.experimental import pallas as pl
from jax.experimental.pallas import tpu as pltpu
from jax.experimental.pallas import tpu_sc as plsc

_B = 16384
_L = 16


@functools.lru_cache(maxsize=None)
def _build(table_len: int):
    info = plsc.get_sparse_core_info()
    nc, ns = 1, info.num_subcores
    nw = nc * ns
    b_per_w = _B // nw

    mesh = plsc.VectorSubcoreMesh(core_axis_name="c", subcore_axis_name="s",
                                  num_cores=nc)

    @functools.partial(
        pl.kernel,
        mesh=mesh,
        out_type=jax.ShapeDtypeStruct((_B,), jnp.float32),
        compiler_params=pltpu.CompilerParams(needs_layout_passes=False),
        scratch_types=[
            pltpu.VMEM((table_len,), jnp.float32),
            pltpu.VMEM((b_per_w,), jnp.int32),
            pltpu.VMEM((b_per_w,), jnp.float32),
            pltpu.SemaphoreType.DMA,
            pltpu.SemaphoreType.DMA,
        ],
    )
    def k(table_hbm, idx_hbm, out_hbm, table_v, idx_v, vals_v, sem_t, sem_i):
        wid = lax.axis_index("s") * nc + lax.axis_index("c")
        base = wid * b_per_w
        cp_t = pltpu.async_copy(table_hbm, table_v, sem_t)
        cp_i = pltpu.async_copy(idx_hbm.at[pl.ds(base, b_per_w)], idx_v, sem_i)
        cp_i.wait()
        cp_t.wait()

        @plsc.parallel_loop(0, b_per_w, _L, unroll=4)
        def _(off):
            idx16 = idx_v[pl.ds(off, _L)]
            vals_v[pl.ds(off, _L)] = plsc.load_gather(table_v, [idx16])
        pltpu.sync_copy(vals_v, out_hbm.at[pl.ds(base, b_per_w)])

    return k


def kernel(betas, t_int):
    return _build(betas.shape[0])(betas.astype(jnp.float32),
                                  t_int.astype(jnp.int32))


# speedup vs baseline: 1.0061x; 1.0016x over previous
import functools

import jax
import jax.numpy as jnp
from jax import lax
from jax.experimental import pallas as pl
from jax.experimental.pallas import tpu as pltpu
from jax.experimental.pallas import tpu_sc as plsc

_B = 16384
_L = 16


@functools.lru_cache(maxsize=None)
def _build(table_len: int):
    info = plsc.get_sparse_core_info()
    nc, ns = 1, info.num_subcores
    nw = nc * ns
    b_per_w = _B // nw

    mesh = plsc.VectorSubcoreMesh(core_axis_name="c", subcore_axis_name="s",
                                  num_cores=nc)

    @functools.partial(
        pl.kernel,
        mesh=mesh,
        out_type=jax.ShapeDtypeStruct((_B,), jnp.float32),
        compiler_params=pltpu.CompilerParams(needs_layout_passes=False),
        scratch_types=[
            pltpu.VMEM((table_len,), jnp.float32),
            pltpu.VMEM((b_per_w,), jnp.int32),
            pltpu.VMEM((b_per_w,), jnp.float32),
            pltpu.SemaphoreType.DMA,
            pltpu.SemaphoreType.DMA,
        ],
    )
    def k(table_hbm, idx_hbm, out_hbm, table_v, idx_v, vals_v, sem_t, sem_i):
        wid = lax.axis_index("s") * nc + lax.axis_index("c")
        base = wid * b_per_w
        cp_t = pltpu.async_copy(table_hbm, table_v, sem_t)
        cp_i = pltpu.async_copy(idx_hbm.at[pl.ds(base, b_per_w)], idx_v, sem_i)
        cp_i.wait()
        cp_t.wait()

        @plsc.parallel_loop(0, b_per_w, _L, unroll=8)
        def _(off):
            idx16 = idx_v[pl.ds(off, _L)]
            vals_v[pl.ds(off, _L)] = plsc.load_gather(table_v, [idx16])
        pltpu.sync_copy(vals_v, out_hbm.at[pl.ds(base, b_per_w)])

    return k


def kernel(betas, t_int):
    return _build(betas.shape[0])(betas.astype(jnp.float32),
                                  t_int.astype(jnp.int32))
